# Initial kernel scaffold; baseline (speedup 1.0000x reference)
import functools

import jax
import jax.numpy as jnp
from jax import lax
from jax.experimental import pallas as pl
from jax.experimental.pallas import tpu as pltpu
from jax.experimental.pallas import tpu_sc as plsc

_B, _N, _D = 8, 4096, 256
_S = 512
_K = 32
_R2 = 0.2 * 0.2

_NC, _NS = 2, 16
_NW = _NC * _NS
_RPW = (_B * _S) // _NW
_IDXCAP = 48


def _fps_body(x_ref, y_ref, z_ref, nx_ref, ny_ref, nz_ref, ci_ref):
    x = x_ref[...]
    y = y_ref[...]
    z = z_ref[...]
    iota_n = lax.broadcasted_iota(jnp.int32, (_B, _N), 1)
    col_iota = lax.broadcasted_iota(jnp.int32, (_B, _S), 1)

    def body(i, st):
        dist, far, nx, ny, nz, cen = st
        onehot = iota_n == far
        cx = jnp.sum(jnp.where(onehot, x, 0.0), axis=1, keepdims=True)
        cy = jnp.sum(jnp.where(onehot, y, 0.0), axis=1, keepdims=True)
        cz = jnp.sum(jnp.where(onehot, z, 0.0), axis=1, keepdims=True)
        sel = col_iota == i
        nx = jnp.where(sel, cx, nx)
        ny = jnp.where(sel, cy, ny)
        nz = jnp.where(sel, cz, nz)
        cen = jnp.where(sel, far, cen)
        dx = x - cx
        dy = y - cy
        dz = z - cz
        d = dx * dx + dy * dy + dz * dz
        dist = jnp.minimum(dist, d)
        m = jnp.max(dist, axis=1, keepdims=True)
        far = jnp.min(jnp.where(dist == m, iota_n, _N), axis=1, keepdims=True)
        return (dist, far, nx, ny, nz, cen)

    init = (
        jnp.full((_B, _N), 1e10, jnp.float32),
        jnp.zeros((_B, 1), jnp.int32),
        jnp.zeros((_B, _S), jnp.float32),
        jnp.zeros((_B, _S), jnp.float32),
        jnp.zeros((_B, _S), jnp.float32),
        jnp.zeros((_B, _S), jnp.int32),
    )
    _, _, nx, ny, nz, cen = lax.fori_loop(0, _S, body, init)
    nx_ref[...] = nx
    ny_ref[...] = ny
    nz_ref[...] = nz
    ci_ref[...] = cen


_fps_call = pl.pallas_call(
    _fps_body,
    out_shape=[
        jax.ShapeDtypeStruct((_B, _S), jnp.float32),
        jax.ShapeDtypeStruct((_B, _S), jnp.float32),
        jax.ShapeDtypeStruct((_B, _S), jnp.float32),
        jax.ShapeDtypeStruct((_B, _S), jnp.int32),
    ],
)


def _sc_body(x_hbm, y_hbm, z_hbm, nx_hbm, ny_hbm, nz_hbm, ci_hbm, feat_hbm,
             out_hbm, xb, yb, zb, nxb, nyb, nzb, cib, idxb, rows, outb, sem):
    c = lax.axis_index("c")
    s = lax.axis_index("s")
    w = s * _NC + c
    row0 = w * _RPW
    b = row0 // _S
    base = b * _N

    pltpu.sync_copy(x_hbm.at[b], xb)
    pltpu.sync_copy(y_hbm.at[b], yb)
    pltpu.sync_copy(z_hbm.at[b], zb)
    pltpu.sync_copy(nx_hbm.at[pl.ds(row0, _RPW)], nxb)
    pltpu.sync_copy(ny_hbm.at[pl.ds(row0, _RPW)], nyb)
    pltpu.sync_copy(nz_hbm.at[pl.ds(row0, _RPW)], nzb)
    pltpu.sync_copy(ci_hbm.at[pl.ds(row0, _RPW)], cib)

    lane = lax.broadcasted_iota(jnp.int32, (16,), 0)

    def row_body(j, carry):
        js = jnp.full((16,), j, jnp.int32)
        cx = plsc.load_gather(nxb, [js])
        cy = plsc.load_gather(nyb, [js])
        cz = plsc.load_gather(nzb, [js])
        pidx = plsc.load_gather(cib, [js]) + base

        idxb[pl.ds(0, 16)] = pidx
        idxb[pl.ds(16, 16)] = pidx
        idxb[pl.ds(32, 16)] = pidx

        def cond(st):
            i, cnt = st
            return jnp.logical_and(i < _N // 16, cnt < _K)

        def sbody(st):
            i, cnt = st
            off = i * 16
            xv = xb[pl.ds(off, 16)]
            yv = yb[pl.ds(off, 16)]
            zv = zb[pl.ds(off, 16)]
            dx = xv - cx
            dy = yv - cy
            dz = zv - cz
            d = dx * dx + dy * dy + dz * dz
            m = d <= _R2
            plsc.store_compressed(idxb.at[pl.ds(cnt, 16)], lane + (off + base),
                                  mask=m)
            cnt = cnt + jnp.sum(m.astype(jnp.int32))
            return (i + 1, cnt)

        lax.while_loop(cond, sbody, (jnp.int32(0), jnp.int32(0)))

        pltpu.async_copy(feat_hbm.at[idxb.at[pl.ds(0, _K)]], rows, sem).wait()

        def mk(kk, accs):
            return tuple(
                jnp.maximum(accs[dc], rows[kk, pl.ds(dc * 16, 16)])
                for dc in range(_D // 16)
            )

        accs = lax.fori_loop(
            1, _K, mk, tuple(rows[0, pl.ds(dc * 16, 16)] for dc in range(_D // 16))
        )
        for dc in range(_D // 16):
            outb[pl.ds(dc * 16, 16)] = accs[dc]
        pltpu.sync_copy(outb, out_hbm.at[row0 + j])
        return carry

    lax.fori_loop(0, _RPW, row_body, 0)


_sc_pool = functools.partial(
    pl.kernel,
    out_type=jax.ShapeDtypeStruct((_B * _S, _D), jnp.float32),
    mesh=plsc.VectorSubcoreMesh(core_axis_name="c", subcore_axis_name="s"),
    scratch_types=[
        pltpu.VMEM((_N,), jnp.float32),
        pltpu.VMEM((_N,), jnp.float32),
        pltpu.VMEM((_N,), jnp.float32),
        pltpu.VMEM((_RPW,), jnp.float32),
        pltpu.VMEM((_RPW,), jnp.float32),
        pltpu.VMEM((_RPW,), jnp.float32),
        pltpu.VMEM((_RPW,), jnp.int32),
        pltpu.VMEM((_IDXCAP,), jnp.int32),
        pltpu.VMEM((_K, _D), jnp.float32),
        pltpu.VMEM((_D,), jnp.float32),
        pltpu.SemaphoreType.DMA,
    ],
)(_sc_body)


@jax.jit
def kernel(xyz, features):
    x = xyz[:, :, 0]
    y = xyz[:, :, 1]
    z = xyz[:, :, 2]
    nx, ny, nz, ci = _fps_call(x, y, z)
    new_xyz = jnp.stack([nx, ny, nz], axis=-1)
    feat_t = jnp.transpose(features, (0, 2, 1)).reshape(_B * _N, _D)
    out = _sc_pool(x, y, z, nx.reshape(-1), ny.reshape(-1), nz.reshape(-1),
                   ci.reshape(-1), feat_t)
    sub_features = jnp.transpose(out.reshape(_B, _S, _D), (0, 2, 1))
    return (new_xyz, sub_features)


# speedup vs baseline: 13.8967x; 13.8967x over previous
import functools

import jax
import jax.numpy as jnp
from jax import lax
from jax.experimental import pallas as pl
from jax.experimental.pallas import tpu as pltpu
from jax.experimental.pallas import tpu_sc as plsc

_B, _N, _D = 8, 4096, 256
_S = 512
_K = 32
_R2 = 0.2 * 0.2

_NC, _NS = 2, 16
_NW = _NC * _NS
_RPW = (_B * _S) // _NW
_IDXCAP = 48


def _fps_body(x_ref, y_ref, z_ref, nx_ref, ny_ref, nz_ref, ci_ref):
    x = x_ref[...]
    y = y_ref[...]
    z = z_ref[...]
    iota_n = lax.broadcasted_iota(jnp.int32, (_B, _N), 1)
    col_iota = lax.broadcasted_iota(jnp.int32, (_B, _S), 1)

    def body(i, st):
        dist, far, nx, ny, nz, cen = st
        onehot = iota_n == far
        cx = jnp.sum(jnp.where(onehot, x, 0.0), axis=1, keepdims=True)
        cy = jnp.sum(jnp.where(onehot, y, 0.0), axis=1, keepdims=True)
        cz = jnp.sum(jnp.where(onehot, z, 0.0), axis=1, keepdims=True)
        sel = col_iota == i
        nx = jnp.where(sel, cx, nx)
        ny = jnp.where(sel, cy, ny)
        nz = jnp.where(sel, cz, nz)
        cen = jnp.where(sel, far, cen)
        dx = x - cx
        dy = y - cy
        dz = z - cz
        d = dx * dx + dy * dy + dz * dz
        dist = jnp.minimum(dist, d)
        m = jnp.max(dist, axis=1, keepdims=True)
        far = jnp.min(jnp.where(dist == m, iota_n, _N), axis=1, keepdims=True)
        return (dist, far, nx, ny, nz, cen)

    row_s = lax.broadcasted_iota(jnp.int32, (_B, _S), 0)
    row_n = lax.broadcasted_iota(jnp.int32, (_B, _N), 0)
    cen0 = col_iota + row_s
    col_f = cen0.astype(jnp.float32)
    init = (
        (iota_n + row_n).astype(jnp.float32) + 1e10,
        jnp.zeros((_B, 1), jnp.int32),
        col_f,
        col_f,
        col_f,
        cen0,
    )
    _, _, nx, ny, nz, cen = lax.fori_loop(0, _S, body, init)
    nx_ref[...] = nx
    ny_ref[...] = ny
    nz_ref[...] = nz
    ci_ref[...] = cen


_fps_call = pl.pallas_call(
    _fps_body,
    out_shape=[
        jax.ShapeDtypeStruct((_B, _S), jnp.float32),
        jax.ShapeDtypeStruct((_B, _S), jnp.float32),
        jax.ShapeDtypeStruct((_B, _S), jnp.float32),
        jax.ShapeDtypeStruct((_B, _S), jnp.int32),
    ],
)


def _sc_body(x_hbm, y_hbm, z_hbm, nx_hbm, ny_hbm, nz_hbm, ci_hbm, feat_hbm,
             out_hbm, xb, yb, zb, nsqb, nxb, nyb, nzb, cib, idxb, rows_a,
             rows_b, outb, sem, sem2):
    c = lax.axis_index("c")
    s = lax.axis_index("s")
    w = s * _NC + c
    row0 = w * _RPW
    b = row0 // _S
    base = b * _N

    pltpu.sync_copy(x_hbm.at[b], xb)
    pltpu.sync_copy(y_hbm.at[b], yb)
    pltpu.sync_copy(z_hbm.at[b], zb)
    pltpu.sync_copy(nx_hbm.at[pl.ds(row0, _RPW)], nxb)
    pltpu.sync_copy(ny_hbm.at[pl.ds(row0, _RPW)], nyb)
    pltpu.sync_copy(nz_hbm.at[pl.ds(row0, _RPW)], nzb)
    pltpu.sync_copy(ci_hbm.at[pl.ds(row0, _RPW)], cib)

    lane = lax.broadcasted_iota(jnp.int32, (16,), 0)

    def bf16_round(v):
        u = plsc.bitcast(v, jnp.int32)
        u = u + (jnp.int32(0x7FFF) + ((u >> 16) & jnp.int32(1)))
        u = u & jnp.int32(-65536)
        return plsc.bitcast(u, jnp.float32)

    def prep_body(i, carry):
        off = i * 16
        xv = xb[pl.ds(off, 16)]
        yv = yb[pl.ds(off, 16)]
        zv = zb[pl.ds(off, 16)]
        nsqb[pl.ds(off, 16)] = (xv * xv + yv * yv) + zv * zv
        xb[pl.ds(off, 16)] = bf16_round(xv)
        yb[pl.ds(off, 16)] = bf16_round(yv)
        zb[pl.ds(off, 16)] = bf16_round(zv)
        return carry

    lax.fori_loop(0, _N // 16, prep_body, 0)

    def row_body(j, carry):
        js = jnp.full((16,), j, jnp.int32)
        cx = plsc.load_gather(nxb, [js])
        cy = plsc.load_gather(nyb, [js])
        cz = plsc.load_gather(nzb, [js])
        pidx = plsc.load_gather(cib, [js]) + base
        ssq = (cx * cx + cy * cy) + cz * cz
        cxb = bf16_round(cx)
        cyb = bf16_round(cy)
        czb = bf16_round(cz)

        idxb[pl.ds(0, 16)] = pidx
        idxb[pl.ds(16, 16)] = pidx
        idxb[pl.ds(32, 16)] = pidx

        def cond(st):
            i, cnt = st
            return jnp.logical_and(i < _N // 16, cnt < _K)

        def sbody(st):
            i, cnt = st
            off = i * 16
            xv = xb[pl.ds(off, 16)]
            yv = yb[pl.ds(off, 16)]
            zv = zb[pl.ds(off, 16)]
            nsqv = nsqb[pl.ds(off, 16)]
            inner = (xv * cxb + yv * cyb) + zv * czb
            d = (jnp.float32(-2.0) * inner + ssq) + nsqv
            m = d <= _R2
            plsc.store_compressed(idxb.at[pl.ds(cnt, 16)], lane + (off + base),
                                  mask=m)
            cnt = cnt + jnp.sum(m.astype(jnp.int32))
            return (i + 1, cnt)

        lax.while_loop(cond, sbody, (jnp.int32(0), jnp.int32(0)))

        ia = idxb[pl.ds(0, 16)]
        ib = idxb[pl.ds(16, 16)]
        da = pltpu.async_copy(feat_hbm.at[ia], rows_a, sem)
        db = pltpu.async_copy(feat_hbm.at[ib], rows_b, sem2)
        da.wait()
        db.wait()

        def mk(kk, accs):
            return tuple(
                jnp.maximum(jnp.maximum(accs[dc], rows_a[kk, pl.ds(dc * 16, 16)]),
                            rows_b[kk, pl.ds(dc * 16, 16)])
                for dc in range(_D // 16)
            )

        accs = lax.fori_loop(
            1, _K // 2, mk,
            tuple(jnp.maximum(rows_a[0, pl.ds(dc * 16, 16)],
                              rows_b[0, pl.ds(dc * 16, 16)])
                  for dc in range(_D // 16)),
        )
        for dc in range(_D // 16):
            outb[pl.ds(dc * 16, 16)] = accs[dc]
        pltpu.sync_copy(outb, out_hbm.at[row0 + j])
        return carry

    lax.fori_loop(0, _RPW, row_body, 0)


@functools.lru_cache(maxsize=None)
def _get_sc_pool():
  return pl.kernel(
    _sc_body,
    out_type=jax.ShapeDtypeStruct((_B * _S, _D), jnp.float32),
    mesh=plsc.VectorSubcoreMesh(core_axis_name="c", subcore_axis_name="s",
                                num_cores=_NC, num_subcores=_NS),
    scratch_types=[
        pltpu.VMEM((_N,), jnp.float32),
        pltpu.VMEM((_N,), jnp.float32),
        pltpu.VMEM((_N,), jnp.float32),
        pltpu.VMEM((_N,), jnp.float32),
        pltpu.VMEM((_RPW,), jnp.float32),
        pltpu.VMEM((_RPW,), jnp.float32),
        pltpu.VMEM((_RPW,), jnp.float32),
        pltpu.VMEM((_RPW,), jnp.int32),
        pltpu.VMEM((_IDXCAP,), jnp.int32),
        pltpu.VMEM((_K // 2, _D), jnp.float32),
        pltpu.VMEM((_K // 2, _D), jnp.float32),
        pltpu.VMEM((_D,), jnp.float32),
        pltpu.SemaphoreType.DMA,
        pltpu.SemaphoreType.DMA,
    ],
    compiler_params=pltpu.CompilerParams(needs_layout_passes=False),
  )


@jax.jit
def kernel(xyz, features):
    x = xyz[:, :, 0]
    y = xyz[:, :, 1]
    z = xyz[:, :, 2]
    nx, ny, nz, ci = _fps_call(x, y, z)
    new_xyz = jnp.stack([nx, ny, nz], axis=-1)
    feat_t = jnp.transpose(features, (0, 2, 1)).reshape(_B * _N, _D)
    out = _get_sc_pool()(x, y, z, nx.reshape(-1), ny.reshape(-1), nz.reshape(-1),
                         ci.reshape(-1), feat_t)
    sub_features = jnp.transpose(out.reshape(_B, _S, _D), (0, 2, 1))
    return (new_xyz, sub_features)


# speedup vs baseline: 19.7804x; 1.4234x over previous
import functools

import jax
import jax.numpy as jnp
from jax import lax
from jax.experimental import pallas as pl
from jax.experimental.pallas import tpu as pltpu
from jax.experimental.pallas import tpu_sc as plsc

_B, _N, _D = 8, 4096, 256
_S = 512
_K = 32
_R2 = 0.2 * 0.2

_NC, _NS = 2, 16
_NW = _NC * _NS
_RPW = (_B * _S) // _NW
_IDXCAP = 80


def _fps_body(x_ref, y_ref, z_ref, nx_ref, ny_ref, nz_ref, ci_ref):
    x = x_ref[...]
    y = y_ref[...]
    z = z_ref[...]
    iota_n = lax.broadcasted_iota(jnp.int32, (_B, _N), 1)
    col_iota = lax.broadcasted_iota(jnp.int32, (_B, _S), 1)

    def body(i, st):
        dist, far, nx, ny, nz, cen = st
        onehot = iota_n == far
        cx = jnp.sum(jnp.where(onehot, x, 0.0), axis=1, keepdims=True)
        cy = jnp.sum(jnp.where(onehot, y, 0.0), axis=1, keepdims=True)
        cz = jnp.sum(jnp.where(onehot, z, 0.0), axis=1, keepdims=True)
        sel = col_iota == i
        nx = jnp.where(sel, cx, nx)
        ny = jnp.where(sel, cy, ny)
        nz = jnp.where(sel, cz, nz)
        cen = jnp.where(sel, far, cen)
        dx = x - cx
        dy = y - cy
        dz = z - cz
        d = dx * dx + dy * dy + dz * dz
        dist = jnp.minimum(dist, d)
        m = jnp.max(dist, axis=1, keepdims=True)
        far = jnp.min(jnp.where(dist == m, iota_n, _N), axis=1, keepdims=True)
        return (dist, far, nx, ny, nz, cen)

    row_s = lax.broadcasted_iota(jnp.int32, (_B, _S), 0)
    row_n = lax.broadcasted_iota(jnp.int32, (_B, _N), 0)
    cen0 = col_iota + row_s
    col_f = cen0.astype(jnp.float32)
    init = (
        (iota_n + row_n).astype(jnp.float32) + 1e10,
        jnp.zeros((_B, 1), jnp.int32),
        col_f,
        col_f,
        col_f,
        cen0,
    )
    _, _, nx, ny, nz, cen = lax.fori_loop(0, _S, body, init)
    nx_ref[...] = nx
    ny_ref[...] = ny
    nz_ref[...] = nz
    ci_ref[...] = cen


_fps_call = pl.pallas_call(
    _fps_body,
    out_shape=[
        jax.ShapeDtypeStruct((_B, _S), jnp.float32),
        jax.ShapeDtypeStruct((_B, _S), jnp.float32),
        jax.ShapeDtypeStruct((_B, _S), jnp.float32),
        jax.ShapeDtypeStruct((_B, _S), jnp.int32),
    ],
)


def _sc_body(x_hbm, y_hbm, z_hbm, nx_hbm, ny_hbm, nz_hbm, ci_hbm, feat_hbm,
             out_hbm, xb, yb, zb, nsqb, nxb, nyb, nzb, cib, idxb,
             r0a, r0b, r1a, r1b, outblk, semg0, semg1):
    c = lax.axis_index("c")
    s = lax.axis_index("s")
    w = s * _NC + c
    b = w // 4
    lane4 = w % 4
    base = b * _N

    pltpu.sync_copy(x_hbm.at[b], xb)
    pltpu.sync_copy(y_hbm.at[b], yb)
    pltpu.sync_copy(z_hbm.at[b], zb)
    pltpu.sync_copy(nx_hbm.at[pl.ds(b * _S, _S)], nxb)
    pltpu.sync_copy(ny_hbm.at[pl.ds(b * _S, _S)], nyb)
    pltpu.sync_copy(nz_hbm.at[pl.ds(b * _S, _S)], nzb)
    pltpu.sync_copy(ci_hbm.at[pl.ds(b * _S, _S)], cib)

    lane = lax.broadcasted_iota(jnp.int32, (16,), 0)

    def bf16_round(v):
        u = plsc.bitcast(v, jnp.int32)
        u = u + (jnp.int32(0x7FFF) + ((u >> 16) & jnp.int32(1)))
        u = u & jnp.int32(-65536)
        return plsc.bitcast(u, jnp.float32)

    def prep_body(i, carry):
        off = i * 16
        xv = xb[pl.ds(off, 16)]
        yv = yb[pl.ds(off, 16)]
        zv = zb[pl.ds(off, 16)]
        nsqb[pl.ds(off, 16)] = (xv * xv + yv * yv) + zv * zv
        xb[pl.ds(off, 16)] = bf16_round(xv)
        yb[pl.ds(off, 16)] = bf16_round(yv)
        zb[pl.ds(off, 16)] = bf16_round(zv)
        return carry

    lax.fori_loop(0, _N // 16, prep_body, 0)

    def scan_row(j):
        js = jnp.full((16,), lane4 + 4 * j, jnp.int32)
        cx = plsc.load_gather(nxb, [js])
        cy = plsc.load_gather(nyb, [js])
        cz = plsc.load_gather(nzb, [js])
        pidx = plsc.load_gather(cib, [js]) + base
        ssq = (cx * cx + cy * cy) + cz * cz
        cxb = bf16_round(cx)
        cyb = bf16_round(cy)
        czb = bf16_round(cz)

        idxb[pl.ds(0, 16)] = pidx
        idxb[pl.ds(16, 16)] = pidx
        idxb[pl.ds(32, 16)] = pidx

        def cond(st):
            i, cnt = st
            return jnp.logical_and(i < _N // 16, cnt < _K)

        def sbody(st):
            i, cnt = st
            for u in range(2):
                off = (i + u) * 16
                xv = xb[pl.ds(off, 16)]
                yv = yb[pl.ds(off, 16)]
                zv = zb[pl.ds(off, 16)]
                nsqv = nsqb[pl.ds(off, 16)]
                inner = (xv * cxb + yv * cyb) + zv * czb
                d = (jnp.float32(-2.0) * inner + ssq) + nsqv
                m = d <= _R2
                plsc.store_compressed(idxb.at[pl.ds(cnt, 16)],
                                      lane + (off + base), mask=m)
                cnt = cnt + jnp.sum(m.astype(jnp.int32))
            return (i + 2, cnt)

        lax.while_loop(cond, sbody, (jnp.int32(0), jnp.int32(0)))

        return idxb[pl.ds(0, 16)], idxb[pl.ds(16, 16)]

    def gather(ia, ib, ra, rb, sem):
        pltpu.async_copy(feat_hbm.at[ia], ra, sem)
        pltpu.async_copy(feat_hbm.at[ib], rb, sem)

    def drain(ra, rb, sem):
        pltpu.make_async_copy(feat_hbm.at[pl.ds(0, _K // 2)], ra, sem).wait()
        pltpu.make_async_copy(feat_hbm.at[pl.ds(0, _K // 2)], rb, sem).wait()

    def maxrow(j, ra, rb):
        def mk(kk, accs):
            return tuple(
                jnp.maximum(jnp.maximum(accs[dc], ra[kk, pl.ds(dc * 16, 16)]),
                            rb[kk, pl.ds(dc * 16, 16)])
                for dc in range(_D // 16)
            )

        accs = lax.fori_loop(
            1, _K // 2, mk,
            tuple(jnp.maximum(ra[0, pl.ds(dc * 16, 16)],
                              rb[0, pl.ds(dc * 16, 16)])
                  for dc in range(_D // 16)),
        )
        for dc in range(_D // 16):
            outblk[j, pl.ds(dc * 16, 16)] = accs[dc]

    ia0, ib0 = scan_row(jnp.int32(0))
    gather(ia0, ib0, r0a, r0b, semg0)

    def pipe_body(g, carry):
        j0 = 2 * g
        j1 = j0 + 1
        ia, ib = scan_row(j1)
        gather(ia, ib, r1a, r1b, semg1)
        drain(r0a, r0b, semg0)
        maxrow(j0, r0a, r0b)

        @pl.when(j1 + 1 < _RPW)
        def _():
            ia2, ib2 = scan_row(j1 + 1)
            gather(ia2, ib2, r0a, r0b, semg0)

        drain(r1a, r1b, semg1)
        maxrow(j1, r1a, r1b)
        return carry

    lax.fori_loop(0, _RPW // 2, pipe_body, 0)
    pltpu.sync_copy(outblk, out_hbm.at[pl.ds(w * _RPW, _RPW)])


@functools.lru_cache(maxsize=None)
def _get_sc_pool():
  return pl.kernel(
    _sc_body,
    out_type=jax.ShapeDtypeStruct((_B * _S, _D), jnp.float32),
    mesh=plsc.VectorSubcoreMesh(core_axis_name="c", subcore_axis_name="s",
                                num_cores=_NC, num_subcores=_NS),
    scratch_types=[
        pltpu.VMEM((_N,), jnp.float32),
        pltpu.VMEM((_N,), jnp.float32),
        pltpu.VMEM((_N,), jnp.float32),
        pltpu.VMEM((_N,), jnp.float32),
        pltpu.VMEM((_S,), jnp.float32),
        pltpu.VMEM((_S,), jnp.float32),
        pltpu.VMEM((_S,), jnp.float32),
        pltpu.VMEM((_S,), jnp.int32),
        pltpu.VMEM((_IDXCAP,), jnp.int32),
        pltpu.VMEM((_K // 2, _D), jnp.float32),
        pltpu.VMEM((_K // 2, _D), jnp.float32),
        pltpu.VMEM((_K // 2, _D), jnp.float32),
        pltpu.VMEM((_K // 2, _D), jnp.float32),
        pltpu.VMEM((_RPW, _D), jnp.float32),
        pltpu.SemaphoreType.DMA,
        pltpu.SemaphoreType.DMA,
    ],
    compiler_params=pltpu.CompilerParams(needs_layout_passes=False),
  )


@jax.jit
def kernel(xyz, features):
    x = xyz[:, :, 0]
    y = xyz[:, :, 1]
    z = xyz[:, :, 2]
    nx, ny, nz, ci = _fps_call(x, y, z)
    new_xyz = jnp.stack([nx, ny, nz], axis=-1)
    feat_t = jnp.transpose(features, (0, 2, 1)).reshape(_B * _N, _D)
    out = _get_sc_pool()(x, y, z, nx.reshape(-1), ny.reshape(-1), nz.reshape(-1),
                         ci.reshape(-1), feat_t)
    out = out.reshape(_B, 4, _RPW, _D).transpose(0, 2, 1, 3).reshape(_B, _S, _D)
    sub_features = jnp.transpose(out, (0, 2, 1))
    return (new_xyz, sub_features)


# speedup vs baseline: 19.8595x; 1.0040x over previous
import functools

import jax
import jax.numpy as jnp
from jax import lax
from jax.experimental import pallas as pl
from jax.experimental.pallas import tpu as pltpu
from jax.experimental.pallas import tpu_sc as plsc

_B, _N, _D = 8, 4096, 256
_S = 512
_K = 32
_R2 = 0.2 * 0.2

_NC, _NS = 2, 16
_NW = _NC * _NS
_RPW = (_B * _S) // _NW
_IDXCAP = 96


def _fps_body(xyz3_ref, nx_ref, ny_ref, nz_ref, ci_ref):
    xyz3 = xyz3_ref[...]
    x = xyz3[0:_B]
    y = xyz3[_B:2 * _B]
    z = xyz3[2 * _B:3 * _B]
    iota_n3 = lax.broadcasted_iota(jnp.int32, (3 * _B, _N), 1)
    col_iota = lax.broadcasted_iota(jnp.int32, (_B, _S), 1)

    def body(i, st):
        dist, far, nx, ny, nz, cen = st
        onehot = iota_n == far
        cx = jnp.max(jnp.where(onehot, x, -1.0), axis=1, keepdims=True)
        cy = jnp.max(jnp.where(onehot, y, -1.0), axis=1, keepdims=True)
        cz = jnp.max(jnp.where(onehot, z, -1.0), axis=1, keepdims=True)
        sel = col_iota == i
        nx = jnp.where(sel, cx, nx)
        ny = jnp.where(sel, cy, ny)
        nz = jnp.where(sel, cz, nz)
        cen = jnp.where(sel, far, cen)
        dx = x - cx
        dy = y - cy
        dz = z - cz
        d = dx * dx + dy * dy + dz * dz
        dist = jnp.minimum(dist, d)
        m = jnp.max(dist, axis=1, keepdims=True)
        far = jnp.min(jnp.where(dist == m, iota_n, _N), axis=1, keepdims=True)
        return (dist, far, nx, ny, nz, cen)

    iota_n = lax.broadcasted_iota(jnp.int32, (_B, _N), 1)
    row_s = lax.broadcasted_iota(jnp.int32, (_B, _S), 0)
    row_n = lax.broadcasted_iota(jnp.int32, (_B, _N), 0)
    cen0 = col_iota + row_s
    col_f = cen0.astype(jnp.float32)
    init = (
        (iota_n + row_n).astype(jnp.float32) + 1e10,
        jnp.zeros((_B, 1), jnp.int32),
        col_f,
        col_f,
        col_f,
        cen0,
    )
    _, _, nx, ny, nz, cen = lax.fori_loop(0, _S, body, init)
    nx_ref[...] = nx
    ny_ref[...] = ny
    nz_ref[...] = nz
    ci_ref[...] = cen


_fps_call = pl.pallas_call(
    _fps_body,
    out_shape=[
        jax.ShapeDtypeStruct((_B, _S), jnp.float32),
        jax.ShapeDtypeStruct((_B, _S), jnp.float32),
        jax.ShapeDtypeStruct((_B, _S), jnp.float32),
        jax.ShapeDtypeStruct((_B, _S), jnp.int32),
    ],
)


def _sc_body(x_hbm, y_hbm, z_hbm, nx_hbm, ny_hbm, nz_hbm, ci_hbm, feat_hbm,
             out_hbm, xb, yb, zb, nsqb, nxb, nyb, nzb, cib, idxb,
             r0a, r0b, r1a, r1b, outblk, semg0, semg1):
    c = lax.axis_index("c")
    s = lax.axis_index("s")
    w = s * _NC + c
    b = w // 4
    lane4 = w % 4
    base = b * _N

    pltpu.sync_copy(x_hbm.at[b], xb)
    pltpu.sync_copy(y_hbm.at[b], yb)
    pltpu.sync_copy(z_hbm.at[b], zb)
    pltpu.sync_copy(nx_hbm.at[pl.ds(b * _S, _S)], nxb)
    pltpu.sync_copy(ny_hbm.at[pl.ds(b * _S, _S)], nyb)
    pltpu.sync_copy(nz_hbm.at[pl.ds(b * _S, _S)], nzb)
    pltpu.sync_copy(ci_hbm.at[pl.ds(b * _S, _S)], cib)

    lane = lax.broadcasted_iota(jnp.int32, (16,), 0)

    def bf16_round(v):
        u = plsc.bitcast(v, jnp.int32)
        u = u + (jnp.int32(0x7FFF) + ((u >> 16) & jnp.int32(1)))
        u = u & jnp.int32(-65536)
        return plsc.bitcast(u, jnp.float32)

    def prep_body(i, carry):
        off = i * 16
        xv = xb[pl.ds(off, 16)]
        yv = yb[pl.ds(off, 16)]
        zv = zb[pl.ds(off, 16)]
        nsqb[pl.ds(off, 16)] = (xv * xv + yv * yv) + zv * zv
        xb[pl.ds(off, 16)] = bf16_round(xv)
        yb[pl.ds(off, 16)] = bf16_round(yv)
        zb[pl.ds(off, 16)] = bf16_round(zv)
        return carry

    lax.fori_loop(0, _N // 16, prep_body, 0)

    def scan_row(j):
        js = jnp.full((16,), lane4 + 4 * j, jnp.int32)
        cx = plsc.load_gather(nxb, [js])
        cy = plsc.load_gather(nyb, [js])
        cz = plsc.load_gather(nzb, [js])
        pidx = plsc.load_gather(cib, [js]) + base
        ssq = (cx * cx + cy * cy) + cz * cz
        cxb = bf16_round(cx)
        cyb = bf16_round(cy)
        czb = bf16_round(cz)

        idxb[pl.ds(0, 16)] = pidx
        idxb[pl.ds(16, 16)] = pidx
        idxb[pl.ds(32, 16)] = pidx

        def cond(st):
            i, cnt_vec = st
            return jnp.logical_and(i < _N // 16, jnp.max(cnt_vec) < _K)

        def sbody(st):
            i, cnt_vec = st
            for u in range(4):
                off = (i + u) * 16
                xv = xb[pl.ds(off, 16)]
                yv = yb[pl.ds(off, 16)]
                zv = zb[pl.ds(off, 16)]
                nsqv = nsqb[pl.ds(off, 16)]
                inner = (xv * cxb + yv * cyb) + zv * czb
                d = (jnp.float32(-2.0) * inner + ssq) + nsqv
                m = d <= _R2
                pos = (cnt_vec - 1) + plsc.cumsum(m.astype(jnp.int32))
                plsc.store_scatter(idxb, [pos], lane + (off + base), mask=m)
                cnt_vec = cnt_vec + plsc.all_reduce_population_count(m)
            return (i + 4, cnt_vec)

        lax.while_loop(cond, sbody, (jnp.int32(0), jnp.zeros((16,), jnp.int32)))

        return idxb[pl.ds(0, 16)], idxb[pl.ds(16, 16)]

    def gather(ia, ib, ra, rb, sem):
        pltpu.async_copy(feat_hbm.at[ia], ra, sem)
        pltpu.async_copy(feat_hbm.at[ib], rb, sem)

    def drain(ra, rb, sem):
        pltpu.make_async_copy(feat_hbm.at[pl.ds(0, _K // 2)], ra, sem).wait()
        pltpu.make_async_copy(feat_hbm.at[pl.ds(0, _K // 2)], rb, sem).wait()

    def maxrow(j, ra, rb):
        def mk(kk, accs):
            return tuple(
                jnp.maximum(jnp.maximum(accs[dc], ra[kk, pl.ds(dc * 16, 16)]),
                            rb[kk, pl.ds(dc * 16, 16)])
                for dc in range(_D // 16)
            )

        accs = lax.fori_loop(
            1, _K // 2, mk,
            tuple(jnp.maximum(ra[0, pl.ds(dc * 16, 16)],
                              rb[0, pl.ds(dc * 16, 16)])
                  for dc in range(_D // 16)),
        )
        for dc in range(_D // 16):
            outblk[j, pl.ds(dc * 16, 16)] = accs[dc]

    ia0, ib0 = scan_row(jnp.int32(0))
    gather(ia0, ib0, r0a, r0b, semg0)

    def pipe_body(g, carry):
        j0 = 2 * g
        j1 = j0 + 1
        ia, ib = scan_row(j1)
        gather(ia, ib, r1a, r1b, semg1)
        drain(r0a, r0b, semg0)
        maxrow(j0, r0a, r0b)

        @pl.when(j1 + 1 < _RPW)
        def _():
            ia2, ib2 = scan_row(j1 + 1)
            gather(ia2, ib2, r0a, r0b, semg0)

        drain(r1a, r1b, semg1)
        maxrow(j1, r1a, r1b)
        return carry

    lax.fori_loop(0, _RPW // 2, pipe_body, 0)
    pltpu.sync_copy(outblk, out_hbm.at[pl.ds(w * _RPW, _RPW)])


@functools.lru_cache(maxsize=None)
def _get_sc_pool():
  return pl.kernel(
    _sc_body,
    out_type=jax.ShapeDtypeStruct((_B * _S, _D), jnp.float32),
    mesh=plsc.VectorSubcoreMesh(core_axis_name="c", subcore_axis_name="s",
                                num_cores=_NC, num_subcores=_NS),
    scratch_types=[
        pltpu.VMEM((_N,), jnp.float32),
        pltpu.VMEM((_N,), jnp.float32),
        pltpu.VMEM((_N,), jnp.float32),
        pltpu.VMEM((_N,), jnp.float32),
        pltpu.VMEM((_S,), jnp.float32),
        pltpu.VMEM((_S,), jnp.float32),
        pltpu.VMEM((_S,), jnp.float32),
        pltpu.VMEM((_S,), jnp.int32),
        pltpu.VMEM((_IDXCAP,), jnp.int32),
        pltpu.VMEM((_K // 2, _D), jnp.float32),
        pltpu.VMEM((_K // 2, _D), jnp.float32),
        pltpu.VMEM((_K // 2, _D), jnp.float32),
        pltpu.VMEM((_K // 2, _D), jnp.float32),
        pltpu.VMEM((_RPW, _D), jnp.float32),
        pltpu.SemaphoreType.DMA,
        pltpu.SemaphoreType.DMA,
    ],
    compiler_params=pltpu.CompilerParams(needs_layout_passes=False),
  )


@jax.jit
def kernel(xyz, features):
    xyz3 = jnp.transpose(xyz, (2, 0, 1))
    x = xyz3[0]
    y = xyz3[1]
    z = xyz3[2]
    nx, ny, nz, ci = _fps_call(xyz3.reshape(3 * _B, _N))
    new_xyz = jnp.stack([nx, ny, nz], axis=-1)
    feat_t = jnp.transpose(features, (0, 2, 1)).reshape(_B * _N, _D)
    out = _get_sc_pool()(x, y, z, nx.reshape(-1), ny.reshape(-1), nz.reshape(-1),
                         ci.reshape(-1), feat_t)
    out = out.reshape(_B, 4, _RPW, _D).transpose(0, 2, 1, 3).reshape(_B, _S, _D)
    sub_features = jnp.transpose(out, (0, 2, 1))
    return (new_xyz, sub_features)


# speedup vs baseline: 28.7604x; 1.4482x over previous
import functools

import jax
import jax.numpy as jnp
from jax import lax
from jax.experimental import pallas as pl
from jax.experimental.pallas import tpu as pltpu
from jax.experimental.pallas import tpu_sc as plsc

_B, _N, _D = 8, 4096, 256
_S = 512
_K = 32
_R2 = 0.2 * 0.2

_NC, _NS = 2, 16
_NW = _NC * _NS
_RPW = (_B * _S) // _NW
_IDXCAP = 160


def _fps_body(xyz3_ref, nx_ref, ny_ref, nz_ref, ci_ref):
    xyz3 = xyz3_ref[...]
    x = xyz3[0:_B]
    y = xyz3[_B:2 * _B]
    z = xyz3[2 * _B:3 * _B]
    col_iota = lax.broadcasted_iota(jnp.int32, (_B, _S), 1)

    def body(i, st):
        dist, far = st
        onehot = iota_n == far
        cx = jnp.max(jnp.where(onehot, x, -1.0), axis=1, keepdims=True)
        cy = jnp.max(jnp.where(onehot, y, -1.0), axis=1, keepdims=True)
        cz = jnp.max(jnp.where(onehot, z, -1.0), axis=1, keepdims=True)
        sel = col_iota == i
        nx_ref[...] = jnp.where(sel, cx, nx_ref[...])
        ny_ref[...] = jnp.where(sel, cy, ny_ref[...])
        nz_ref[...] = jnp.where(sel, cz, nz_ref[...])
        ci_ref[...] = jnp.where(sel, far, ci_ref[...])
        dx = x - cx
        dy = y - cy
        dz = z - cz
        d = dx * dx + dy * dy + dz * dz
        dist = jnp.minimum(dist, d)
        m = jnp.max(dist, axis=1, keepdims=True)
        far = jnp.min(jnp.where(dist == m, iota_n, _N), axis=1, keepdims=True)
        return (dist, far)

    iota_n = lax.broadcasted_iota(jnp.int32, (_B, _N), 1)
    row_n = lax.broadcasted_iota(jnp.int32, (_B, _N), 0)
    init = (
        (iota_n + row_n).astype(jnp.float32) + 1e10,
        jnp.zeros((_B, 1), jnp.int32),
    )
    lax.fori_loop(0, _S, body, init)


_fps_call = pl.pallas_call(
    _fps_body,
    out_shape=[
        jax.ShapeDtypeStruct((_B, _S), jnp.float32),
        jax.ShapeDtypeStruct((_B, _S), jnp.float32),
        jax.ShapeDtypeStruct((_B, _S), jnp.float32),
        jax.ShapeDtypeStruct((_B, _S), jnp.int32),
    ],
)


def _sc_body(x_hbm, y_hbm, z_hbm, nx_hbm, ny_hbm, nz_hbm, ci_hbm, feat_hbm,
             out_hbm, xb, yb, zb, nsqb, nxb, nyb, nzb, cib, idxb,
             ig0a, ig0b, ig1a, ig1b, r0a, r0b, r1a, r1b, outblk, semg0,
             semg1):
    c = lax.axis_index("c")
    s = lax.axis_index("s")
    w = s * _NC + c
    b = w // 4
    lane4 = w % 4
    base = b * _N

    pltpu.sync_copy(x_hbm.at[b], xb)
    pltpu.sync_copy(y_hbm.at[b], yb)
    pltpu.sync_copy(z_hbm.at[b], zb)
    pltpu.sync_copy(nx_hbm.at[pl.ds(b * _S, _S)], nxb)
    pltpu.sync_copy(ny_hbm.at[pl.ds(b * _S, _S)], nyb)
    pltpu.sync_copy(nz_hbm.at[pl.ds(b * _S, _S)], nzb)
    pltpu.sync_copy(ci_hbm.at[pl.ds(b * _S, _S)], cib)

    lane = lax.broadcasted_iota(jnp.int32, (16,), 0)

    def bf16_round(v):
        u = plsc.bitcast(v, jnp.int32)
        u = u + (jnp.int32(0x7FFF) + ((u >> 16) & jnp.int32(1)))
        u = u & jnp.int32(-65536)
        return plsc.bitcast(u, jnp.float32)

    def prep_body(i, carry):
        off = i * 16
        xv = xb[pl.ds(off, 16)]
        yv = yb[pl.ds(off, 16)]
        zv = zb[pl.ds(off, 16)]
        nsqb[pl.ds(off, 16)] = (xv * xv + yv * yv) + zv * zv
        xb[pl.ds(off, 16)] = bf16_round(xv)
        yb[pl.ds(off, 16)] = bf16_round(yv)
        zb[pl.ds(off, 16)] = bf16_round(zv)
        return carry

    lax.fori_loop(0, _N // 16, prep_body, 0)

    def scan_row(j):
        js = jnp.full((16,), lane4 + 4 * j, jnp.int32)
        cx = plsc.load_gather(nxb, [js])
        cy = plsc.load_gather(nyb, [js])
        cz = plsc.load_gather(nzb, [js])
        pidx = plsc.load_gather(cib, [js]) + base
        ssq = (cx * cx + cy * cy) + cz * cz
        cxb = bf16_round(cx)
        cyb = bf16_round(cy)
        czb = bf16_round(cz)

        idxb[pl.ds(0, 16)] = pidx
        idxb[pl.ds(16, 16)] = pidx
        idxb[pl.ds(32, 16)] = pidx

        def cond(st):
            i, cnt_vec = st
            return jnp.logical_and(i < _N // 16, jnp.max(cnt_vec) < _K)

        def sbody(st):
            i, cnt_vec = st
            U = 4
            for rep in range(2):
                offs = [(i + rep * U + u) * 16 for u in range(U)]
                xs = [xb[pl.ds(o, 16)] for o in offs]
                ys = [yb[pl.ds(o, 16)] for o in offs]
                zs = [zb[pl.ds(o, 16)] for o in offs]
                ns = [nsqb[pl.ds(o, 16)] for o in offs]
                t1 = [xs[u] * cxb for u in range(U)]
                t2 = [ys[u] * cyb for u in range(U)]
                t3 = [zs[u] * czb for u in range(U)]
                inn = [(t1[u] + t2[u]) + t3[u] for u in range(U)]
                ds = [(jnp.float32(-2.0) * inn[u] + ssq) + ns[u]
                      for u in range(U)]
                ms = [ds[u] <= _R2 for u in range(U)]
                rks = [plsc.cumsum(ms[u].astype(jnp.int32)) for u in range(U)]
                pcs = [plsc.all_reduce_population_count(ms[u])
                       for u in range(U)]
                cbase = []
                for u in range(U):
                    cbase.append(cnt_vec)
                    cnt_vec = cnt_vec + pcs[u]
                for u in range(U):
                    pos = (cbase[u] - 1) + rks[u]
                    plsc.store_scatter(idxb, [pos], lane + (offs[u] + base),
                                       mask=ms[u])
            return (i + 2 * U, cnt_vec)

        lax.while_loop(cond, sbody, (jnp.int32(0), jnp.zeros((16,), jnp.int32)))

    def stage_idx(iga, igb):
        iga[...] = idxb[pl.ds(0, 16)]
        igb[...] = idxb[pl.ds(16, 16)]

    def gather(iga, igb, ra, rb, sem):
        pltpu.async_copy(feat_hbm.at[iga], ra, sem)
        pltpu.async_copy(feat_hbm.at[igb], rb, sem)

    def drain(ra, rb, sem):
        pltpu.make_async_copy(feat_hbm.at[pl.ds(0, _K // 2)], ra, sem).wait()
        pltpu.make_async_copy(feat_hbm.at[pl.ds(0, _K // 2)], rb, sem).wait()

    def maxrow(j, ra, rb):
        def mk(kk, accs):
            return tuple(
                jnp.maximum(jnp.maximum(accs[dc], ra[kk, pl.ds(dc * 16, 16)]),
                            rb[kk, pl.ds(dc * 16, 16)])
                for dc in range(_D // 16)
            )

        accs = lax.fori_loop(
            1, _K // 2, mk,
            tuple(jnp.maximum(ra[0, pl.ds(dc * 16, 16)],
                              rb[0, pl.ds(dc * 16, 16)])
                  for dc in range(_D // 16)),
        )
        for dc in range(_D // 16):
            outblk[j, pl.ds(dc * 16, 16)] = accs[dc]

    scan_row(jnp.int32(0))
    stage_idx(ig0a, ig0b)
    gather(ig0a, ig0b, r0a, r0b, semg0)

    def pipe_body(g, carry):
        j0 = 2 * g
        j1 = j0 + 1
        scan_row(j1)
        stage_idx(ig1a, ig1b)
        gather(ig1a, ig1b, r1a, r1b, semg1)
        drain(r0a, r0b, semg0)
        maxrow(j0, r0a, r0b)

        @pl.when(j1 + 1 < _RPW)
        def _():
            scan_row(j1 + 1)
            stage_idx(ig0a, ig0b)
            gather(ig0a, ig0b, r0a, r0b, semg0)

        drain(r1a, r1b, semg1)
        maxrow(j1, r1a, r1b)
        return carry

    lax.fori_loop(0, _RPW // 2, pipe_body, 0)
    pltpu.sync_copy(outblk, out_hbm.at[pl.ds(w * _RPW, _RPW)])


@functools.lru_cache(maxsize=None)
def _get_sc_pool():
  return pl.kernel(
    _sc_body,
    out_type=jax.ShapeDtypeStruct((_B * _S, _D), jnp.float32),
    mesh=plsc.VectorSubcoreMesh(core_axis_name="c", subcore_axis_name="s",
                                num_cores=_NC, num_subcores=_NS),
    scratch_types=[
        pltpu.VMEM((_N,), jnp.float32),
        pltpu.VMEM((_N,), jnp.float32),
        pltpu.VMEM((_N,), jnp.float32),
        pltpu.VMEM((_N,), jnp.float32),
        pltpu.VMEM((_S,), jnp.float32),
        pltpu.VMEM((_S,), jnp.float32),
        pltpu.VMEM((_S,), jnp.float32),
        pltpu.VMEM((_S,), jnp.int32),
        pltpu.VMEM((_IDXCAP,), jnp.int32),
        pltpu.VMEM((16,), jnp.int32),
        pltpu.VMEM((16,), jnp.int32),
        pltpu.VMEM((16,), jnp.int32),
        pltpu.VMEM((16,), jnp.int32),
        pltpu.VMEM((_K // 2, _D), jnp.float32),
        pltpu.VMEM((_K // 2, _D), jnp.float32),
        pltpu.VMEM((_K // 2, _D), jnp.float32),
        pltpu.VMEM((_K // 2, _D), jnp.float32),
        pltpu.VMEM((_RPW, _D), jnp.float32),
        pltpu.SemaphoreType.DMA,
        pltpu.SemaphoreType.DMA,
    ],
    compiler_params=pltpu.CompilerParams(needs_layout_passes=False),
  )


@jax.jit
def kernel(xyz, features):
    xyz3 = jnp.transpose(xyz, (2, 0, 1))
    x = xyz3[0]
    y = xyz3[1]
    z = xyz3[2]
    nx, ny, nz, ci = _fps_call(xyz3.reshape(3 * _B, _N))
    new_xyz = jnp.stack([nx, ny, nz], axis=-1)
    feat_t = jnp.transpose(features, (0, 2, 1)).reshape(_B * _N, _D)
    out = _get_sc_pool()(x, y, z, nx.reshape(-1), ny.reshape(-1), nz.reshape(-1),
                         ci.reshape(-1), feat_t)
    out = out.reshape(_B, 4, _RPW, _D).transpose(0, 2, 1, 3).reshape(_B, _S, _D)
    sub_features = jnp.transpose(out, (0, 2, 1))
    return (new_xyz, sub_features)
